# Initial kernel scaffold; baseline (speedup 1.0000x reference)
import jax
import jax.numpy as jnp
from jax.experimental import pallas as pl


def kernel(x, edge_index, W1_0, W2_0, W3_0, W4_0, W1_1, W2_1, W3_1, W4_1, W1_2, W2_2, W3_2, W4_2):
    raise NotImplementedError("write your pallas kernel here")


# speedup vs baseline: 11.5447x; 11.5447x over previous
import functools

import jax
import jax.numpy as jnp
from jax import lax
from jax.experimental import pallas as pl
from jax.experimental.pallas import tpu as pltpu
from jax.experimental.pallas import tpu_sc as plsc

N = 10000
E = 320000
D = 128

NC = 2
NS = 16
NW = NC * NS

C = 125
KTOT = E // C
K = KTOT // NW
K2 = KTOT // NS
DH = D // 2
RT = 632
NA = RT * NS
RC0, RC1 = 320, RT - 320

BR = 1000
GRID = N // BR

_mesh = plsc.VectorSubcoreMesh(
    core_axis_name="c", subcore_axis_name="s", num_cores=NC, num_subcores=NS
)


def _deg_body(row2, ones_hbm, zer_hbm, out_hbm, idx_r, ones_v, zbuf, acc):
    c = lax.axis_index("c")
    s = lax.axis_index("s")
    base = (c * NS + s) * K
    pltpu.sync_copy(row2.at[pl.ds(base, K)], idx_r)
    pltpu.sync_copy(ones_hbm, ones_v)
    pltpu.sync_copy(zer_hbm, zbuf)
    pltpu.sync_copy(zbuf.at[pl.ds(0, RC0)], acc.at[pl.ds(s * RT, RC0)])
    pltpu.sync_copy(zbuf.at[pl.ds(0, RC1)], acc.at[pl.ds(s * RT + RC0, RC1)])
    plsc.subcore_barrier()

    def step(j, carry):
        pltpu.sync_copy(ones_v, acc.at[idx_r.at[j]], add=True)
        return carry

    lax.fori_loop(0, K, step, 0)
    plsc.subcore_barrier()
    obase = c * NA + s * RT
    pltpu.sync_copy(acc.at[pl.ds(s * RT, RC0)], zbuf.at[pl.ds(0, RC0)])
    pltpu.sync_copy(zbuf.at[pl.ds(0, RC0)], out_hbm.at[pl.ds(obase, RC0)])
    pltpu.sync_copy(acc.at[pl.ds(s * RT + RC0, RC1)], zbuf.at[pl.ds(0, RC1)])
    pltpu.sync_copy(zbuf.at[pl.ds(0, RC1)], out_hbm.at[pl.ds(obase + RC0, RC1)])


def _seg_body(tab2, row2, col3, zer_hbm, out_hbm, idx_r, idx_c, rows_v, obuf, acc, sem):
    c = lax.axis_index("c")
    s = lax.axis_index("s")
    base = s * K2
    pltpu.sync_copy(row2.at[pl.ds(base, K2)], idx_r)
    pltpu.sync_copy(col3.at[pl.ds(c * KTOT + base, K2)], idx_c)
    pltpu.sync_copy(zer_hbm, obuf)
    pltpu.sync_copy(obuf.at[pl.ds(0, RC0)], acc.at[pl.ds(s * RT, RC0)])
    pltpu.sync_copy(obuf.at[pl.ds(0, RC1)], acc.at[pl.ds(s * RT + RC0, RC1)])
    plsc.subcore_barrier()

    def step(j, carry):
        pltpu.async_copy(tab2.at[idx_c.at[j]], rows_v, sem).wait()
        pltpu.sync_copy(rows_v, acc.at[idx_r.at[j]], add=True)
        return carry

    lax.fori_loop(0, K2, step, 0)
    plsc.subcore_barrier()
    obase = c * NA + s * RT
    pltpu.sync_copy(acc.at[pl.ds(s * RT, RC0)], obuf.at[pl.ds(0, RC0)])
    pltpu.sync_copy(obuf.at[pl.ds(0, RC0)], out_hbm.at[pl.ds(obase, RC0)])
    pltpu.sync_copy(acc.at[pl.ds(s * RT + RC0, RC1)], obuf.at[pl.ds(0, RC1)])
    pltpu.sync_copy(obuf.at[pl.ds(0, RC1)], out_hbm.at[pl.ds(obase + RC0, RC1)])


_deg_pass = pl.kernel(
    _deg_body,
    out_type=jax.ShapeDtypeStruct((NC * NA, 16), jnp.float32),
    mesh=_mesh,
    compiler_params=pltpu.CompilerParams(use_tc_tiling_on_sc=False),
    scratch_types=[
        pltpu.VMEM((K, C), jnp.int32),
        pltpu.VMEM((C, 16), jnp.float32),
        pltpu.VMEM((RC0, 16), jnp.float32),
        pltpu.VMEM_SHARED((NA, 16), jnp.float32),
    ],
)

_seg_pass = pl.kernel(
    _seg_body,
    out_type=jax.ShapeDtypeStruct((NC * NA, DH), jnp.float32),
    mesh=_mesh,
    compiler_params=pltpu.CompilerParams(use_tc_tiling_on_sc=False),
    scratch_types=[
        pltpu.VMEM((K2, C), jnp.int32),
        pltpu.VMEM((K2, C), jnp.int32),
        pltpu.VMEM((C, DH), jnp.float32),
        pltpu.VMEM((RC0, DH), jnp.float32),
        pltpu.VMEM_SHARED((NA, DH), jnp.float32),
        pltpu.SemaphoreType.DMA,
    ],
)


def _mm_t(a, b):
    return lax.dot_general(a, b, (((1,), (1,)), ((), ())),
                           preferred_element_type=jnp.float32)


def _v3_row(W3_ref, W4_ref):
    r4 = jax.nn.relu(W4_ref[...])
    return lax.dot_general(r4, W3_ref[...], (((0,), (1,)), ((), ())),
                           preferred_element_type=jnp.float32)


def _stage0_body(x_ref, dp_ref, W1_ref, W3a_ref, W4a_ref, W2b_ref, W3b_ref, W4b_ref, o_ref):
    deg = dp_ref[0, :, 0:1] + dp_ref[1, :, 0:1]
    h1 = jax.nn.relu(_mm_t(x_ref[...], W1_ref[...]) + deg * _v3_row(W3a_ref, W4a_ref))
    g = _mm_t(h1, W2b_ref[...]) + _v3_row(W3b_ref, W4b_ref)
    o_ref[0] = g[:, :DH]
    o_ref[1] = g[:, DH:]


def _stage1_body(x_ref, p_ref, W1_ref, W2b_ref, W3b_ref, W4b_ref, o_ref):
    seg = jnp.concatenate([p_ref[0], p_ref[1]], axis=1)
    h = jax.nn.relu(_mm_t(x_ref[...], W1_ref[...]) + seg)
    g = _mm_t(h, W2b_ref[...]) + _v3_row(W3b_ref, W4b_ref)
    o_ref[0] = g[:, :DH]
    o_ref[1] = g[:, DH:]


def _stage2_body(x_ref, p_ref, W1_ref, o_ref):
    seg = jnp.concatenate([p_ref[0], p_ref[1]], axis=1)
    o_ref[...] = jax.nn.relu(_mm_t(x_ref[...], W1_ref[...]) + seg)


_xspec = pl.BlockSpec((BR, D), lambda i: (i, 0))
_wspec = pl.BlockSpec((D, D), lambda i: (0, 0))
_w4spec = pl.BlockSpec((D, 1), lambda i: (0, 0))
_ospec = pl.BlockSpec((BR, D), lambda i: (i, 0))


_gspec = pl.BlockSpec((NC, BR, DH), lambda i: (0, i, 0))

_stage0 = pl.pallas_call(
    _stage0_body,
    grid=(GRID,),
    in_specs=[_xspec, pl.BlockSpec((NC, BR, 16), lambda i: (0, i, 0)),
              _wspec, _wspec, _w4spec, _wspec, _wspec, _w4spec],
    out_specs=_gspec,
    out_shape=jax.ShapeDtypeStruct((NC, N, DH), jnp.float32),
)

_stage1 = pl.pallas_call(
    _stage1_body,
    grid=(GRID,),
    in_specs=[_xspec, pl.BlockSpec((NC, BR, DH), lambda i: (0, i, 0)),
              _wspec, _wspec, _wspec, _w4spec],
    out_specs=_gspec,
    out_shape=jax.ShapeDtypeStruct((NC, N, DH), jnp.float32),
)

_stage2 = pl.pallas_call(
    _stage2_body,
    grid=(GRID,),
    in_specs=[_xspec, pl.BlockSpec((NC, BR, DH), lambda i: (0, i, 0)), _wspec],
    out_specs=_ospec,
    out_shape=jax.ShapeDtypeStruct((N, D), jnp.float32),
)


def kernel(x, edge_index, W1_0, W2_0, W3_0, W4_0, W1_1, W2_1, W3_1, W4_1,
           W1_2, W2_2, W3_2, W4_2):
    row2 = edge_index[0].reshape(KTOT, C)
    col2 = edge_index[1].reshape(KTOT, C)
    col3 = jnp.concatenate([col2, col2 + N], axis=0)
    zeros = jnp.zeros((RC0, DH), jnp.float32)
    zeros16 = jnp.zeros((RC0, 16), jnp.float32)
    ones16 = jnp.ones((C, 16), jnp.float32)

    dp = _deg_pass(row2, ones16, zeros16).reshape(NC, NA, 16)
    g1 = _stage0(x, dp, W1_0, W3_0, W4_0, W2_1, W3_1, W4_1)
    p1 = _seg_pass(g1.reshape(NC * N, DH), row2, col3, zeros)
    g2 = _stage1(x, p1.reshape(NC, NA, DH), W1_1, W2_2, W3_2, W4_2)
    p2 = _seg_pass(g2.reshape(NC * N, DH), row2, col3, zeros)
    return _stage2(x, p2.reshape(NC, NA, DH), W1_2)


# speedup vs baseline: 19.3316x; 1.6745x over previous
import jax
import jax.numpy as jnp
from jax import lax
from jax.experimental import pallas as pl
from jax.experimental.pallas import tpu as pltpu
from jax.experimental.pallas import tpu_sc as plsc

N = 10000
E = 320000
D = 128

NC = 2
NS = 16
NW = NC * NS

C = 125
KTOT = E // C
K = KTOT // NW
K2 = KTOT // NS
DH = D // 2
RT = 632
NA = RT * NS
RC0, RC1 = 320, RT - 320

BR = 1000
GRID = N // BR

_mesh = plsc.VectorSubcoreMesh(
    core_axis_name="c", subcore_axis_name="s", num_cores=NC, num_subcores=NS
)


def _deg_body(ei2, ones_hbm, zer_hbm, out_hbm, idx_r, ones_v, zbuf, acc):
    c = lax.axis_index("c")
    s = lax.axis_index("s")
    base = (c * NS + s) * K
    pltpu.sync_copy(ei2.at[pl.ds(base, K)], idx_r)
    pltpu.sync_copy(ones_hbm, ones_v)
    pltpu.sync_copy(zer_hbm, zbuf)
    pltpu.sync_copy(zbuf.at[pl.ds(0, RC0)], acc.at[pl.ds(s * RT, RC0)])
    pltpu.sync_copy(zbuf.at[pl.ds(0, RC1)], acc.at[pl.ds(s * RT + RC0, RC1)])
    plsc.subcore_barrier()

    def step(j, carry):
        pltpu.sync_copy(ones_v, acc.at[idx_r.at[j]], add=True)
        return carry

    lax.fori_loop(0, K, step, 0)
    plsc.subcore_barrier()
    obase = c * NA + s * RT
    pltpu.sync_copy(acc.at[pl.ds(s * RT, RC0)], zbuf.at[pl.ds(0, RC0)])
    pltpu.sync_copy(zbuf.at[pl.ds(0, RC0)], out_hbm.at[pl.ds(obase, RC0)])
    pltpu.sync_copy(acc.at[pl.ds(s * RT + RC0, RC1)], zbuf.at[pl.ds(0, RC1)])
    pltpu.sync_copy(zbuf.at[pl.ds(0, RC1)], out_hbm.at[pl.ds(obase + RC0, RC1)])


def _seg_body(tab2, ei2, zer_hbm, out_hbm, idx_r, idx_c,
              rows_a, rows_b, rows_c, rows_d, rows_e, acc,
              sem_ga, sem_gb, sem_gc, sem_gd, sem_ge,
              sem_sa, sem_sb, sem_sc, sem_sd, sem_se):
    c = lax.axis_index("c")
    s = lax.axis_index("s")
    base = s * K2
    pltpu.sync_copy(ei2.at[pl.ds(base, K2)], idx_r)
    pltpu.sync_copy(ei2.at[pl.ds(KTOT + base, K2)], idx_c)
    tab_c = tab2.at[pl.ds(c * N, N)]
    pltpu.sync_copy(zer_hbm, acc.at[pl.ds(s * RT, RC0)])
    pltpu.sync_copy(zer_hbm.at[pl.ds(0, RC1)], acc.at[pl.ds(s * RT + RC0, RC1)])
    plsc.subcore_barrier()

    rows = (rows_a, rows_b, rows_c, rows_d, rows_e)
    sg = (sem_ga, sem_gb, sem_gc, sem_gd, sem_ge)
    ss = (sem_sa, sem_sb, sem_sc, sem_sd, sem_se)
    NB = 5

    def gather(j, b):
        pltpu.async_copy(tab_c.at[idx_c.at[j]], rows[b], sg[b])

    def gwait(j, b):
        pltpu.make_async_copy(tab_c.at[idx_c.at[j]], rows[b], sg[b]).wait()

    def scatter(j, b):
        pltpu.async_copy(rows[b], acc.at[idx_r.at[j]], ss[b], add=True)

    def swait(j, b):
        pltpu.make_async_copy(rows[b], acc.at[idx_r.at[j]], ss[b]).wait()

    for b in range(NB):
        gather(b, b)

    def step(i, carry):
        j = i * NB
        for b in range(NB):
            gwait(j + b, b)
            scatter(j + b, b)
        for b in range(NB):
            swait(j + b, b)
            gather(j + NB + b, b)
        return carry

    lax.fori_loop(0, K2 // NB - 1, step, 0)
    jl = K2 - NB
    for b in range(NB):
        gwait(jl + b, b)
        scatter(jl + b, b)
    for b in range(NB):
        swait(jl + b, b)
    plsc.subcore_barrier()
    obase = c * NA + s * RT
    pltpu.sync_copy(acc.at[pl.ds(s * RT, RT)], out_hbm.at[pl.ds(obase, RT)])


_deg_pass = pl.kernel(
    _deg_body,
    out_type=jax.ShapeDtypeStruct((NC * NA, 16), jnp.float32),
    mesh=_mesh,
    compiler_params=pltpu.CompilerParams(use_tc_tiling_on_sc=False),
    scratch_types=[
        pltpu.VMEM((K, C), jnp.int32),
        pltpu.VMEM((C, 16), jnp.float32),
        pltpu.VMEM((RC0, 16), jnp.float32),
        pltpu.VMEM_SHARED((NA, 16), jnp.float32),
    ],
)

_seg_pass = pl.kernel(
    _seg_body,
    out_type=jax.ShapeDtypeStruct((NC * NA, DH), jnp.float32),
    mesh=_mesh,
    compiler_params=pltpu.CompilerParams(use_tc_tiling_on_sc=False),
    scratch_types=[
        pltpu.VMEM((K2, C), jnp.int32),
        pltpu.VMEM((K2, C), jnp.int32),
        pltpu.VMEM((C, DH), jnp.float32),
        pltpu.VMEM((C, DH), jnp.float32),
        pltpu.VMEM((C, DH), jnp.float32),
        pltpu.VMEM((C, DH), jnp.float32),
        pltpu.VMEM((C, DH), jnp.float32),
        pltpu.VMEM_SHARED((NA, DH), jnp.float32),
    ] + [pltpu.SemaphoreType.DMA] * 10,
)


def _mm_t(a, b):
    return lax.dot_general(a, b, (((1,), (1,)), ((), ())),
                           preferred_element_type=jnp.float32)


def _v3_row(W3_ref, W4_ref):
    r4 = jax.nn.relu(W4_ref[...])
    return lax.dot_general(r4, W3_ref[...], (((0,), (1,)), ((), ())),
                           preferred_element_type=jnp.float32)


def _stage0_body(x_ref, dp_ref, W1_ref, W3a_ref, W4a_ref, W2b_ref, W3b_ref, W4b_ref, o_ref):
    deg = dp_ref[0, :, 0:1] + dp_ref[1, :, 0:1]
    h1 = jax.nn.relu(_mm_t(x_ref[...], W1_ref[...]) + deg * _v3_row(W3a_ref, W4a_ref))
    g = _mm_t(h1, W2b_ref[...]) + _v3_row(W3b_ref, W4b_ref)
    o_ref[0] = g[:, :DH]
    o_ref[1] = g[:, DH:]


def _stage1_body(x_ref, p_ref, W1_ref, W2b_ref, W3b_ref, W4b_ref, o_ref):
    seg = jnp.concatenate([p_ref[0], p_ref[1]], axis=1)
    h = jax.nn.relu(_mm_t(x_ref[...], W1_ref[...]) + seg)
    g = _mm_t(h, W2b_ref[...]) + _v3_row(W3b_ref, W4b_ref)
    o_ref[0] = g[:, :DH]
    o_ref[1] = g[:, DH:]


def _stage2_body(x_ref, p_ref, W1_ref, o_ref):
    seg = jnp.concatenate([p_ref[0], p_ref[1]], axis=1)
    o_ref[...] = jax.nn.relu(_mm_t(x_ref[...], W1_ref[...]) + seg)


_xspec = pl.BlockSpec((BR, D), lambda i: (i, 0))
_wspec = pl.BlockSpec((D, D), lambda i: (0, 0))
_w4spec = pl.BlockSpec((D, 1), lambda i: (0, 0))
_ospec = pl.BlockSpec((BR, D), lambda i: (i, 0))


_gspec = pl.BlockSpec((NC, BR, DH), lambda i: (0, i, 0))

_stage0 = pl.pallas_call(
    _stage0_body,
    grid=(GRID,),
    in_specs=[_xspec, pl.BlockSpec((NC, BR, 16), lambda i: (0, i, 0)),
              _wspec, _wspec, _w4spec, _wspec, _wspec, _w4spec],
    out_specs=_gspec,
    out_shape=jax.ShapeDtypeStruct((NC, N, DH), jnp.float32),
)

_stage1 = pl.pallas_call(
    _stage1_body,
    grid=(GRID,),
    in_specs=[_xspec, pl.BlockSpec((NC, BR, DH), lambda i: (0, i, 0)),
              _wspec, _wspec, _wspec, _w4spec],
    out_specs=_gspec,
    out_shape=jax.ShapeDtypeStruct((NC, N, DH), jnp.float32),
)

_stage2 = pl.pallas_call(
    _stage2_body,
    grid=(GRID,),
    in_specs=[_xspec, pl.BlockSpec((NC, BR, DH), lambda i: (0, i, 0)), _wspec],
    out_specs=_ospec,
    out_shape=jax.ShapeDtypeStruct((N, D), jnp.float32),
)


def kernel(x, edge_index, W1_0, W2_0, W3_0, W4_0, W1_1, W2_1, W3_1, W4_1,
           W1_2, W2_2, W3_2, W4_2):
    ei2 = edge_index.reshape(2 * KTOT, C)
    zeros = jnp.zeros((RC0, DH), jnp.float32)
    zeros16 = jnp.zeros((RC0, 16), jnp.float32)
    ones16 = jnp.ones((C, 16), jnp.float32)

    dp = _deg_pass(ei2, ones16, zeros16).reshape(NC, NA, 16)
    g1 = _stage0(x, dp, W1_0, W3_0, W4_0, W2_1, W3_1, W4_1)
    p1 = _seg_pass(g1.reshape(NC * N, DH), ei2, zeros)
    g2 = _stage1(x, p1.reshape(NC, NA, DH), W1_1, W2_2, W3_2, W4_2)
    p2 = _seg_pass(g2.reshape(NC * N, DH), ei2, zeros)
    return _stage2(x, p2.reshape(NC, NA, DH), W1_2)
